# baseline (speedup 1.0000x reference)
import functools

import jax
import jax.numpy as jnp
from jax import lax
from jax.experimental import pallas as pl
from jax.experimental.pallas import tpu as pltpu
from jax.experimental.pallas import tpu_sc as plsc

_B = 16384
_H = 1024
_X = 2624
_D = _H + _H + _X
_NW = 32
_RPW = _B // _NW
_R = 8
_NCH = _RPW // _R


def _sc_body(c_hbm, h_hbm, x_hbm, w0_hbm, w1_hbm, w2_hbm, b0_hbm, b1_hbm, b2_hbm,
             o0_hbm, o1_hbm, o2_hbm,
             cb0, hb0, xb0, cb1, hb1, xb1,
             w0, w1, w2, bv0, bv1, bv2,
             ob0, ob1, ob2,
             cs0, hs0, xs0, cs1, hs1, xs1, wsem):
    wid = lax.axis_index("s") * 2 + lax.axis_index("c")
    base = wid * _RPW

    pltpu.async_copy(w0_hbm, w0, wsem).wait()
    pltpu.async_copy(w1_hbm, w1, wsem).wait()
    pltpu.async_copy(w2_hbm, w2, wsem).wait()
    pltpu.async_copy(b0_hbm, bv0, wsem).wait()
    pltpu.async_copy(b1_hbm, bv1, wsem).wait()
    pltpu.async_copy(b2_hbm, bv2, wsem).wait()

    bufs = ((cb0, hb0, xb0, cs0, hs0, xs0),
            (cb1, hb1, xb1, cs1, hs1, xs1))

    def start(slot, ch):
        cb, hb, xb, cs, hs, xs = bufs[slot]
        row = base + ch * _R
        pltpu.make_async_copy(
            c_hbm.at[pl.ds(row, _R), :], cb, cs).start()
        pltpu.make_async_copy(
            h_hbm.at[pl.ds(row, _R), :], hb, hs).start()
        pltpu.make_async_copy(
            x_hbm.at[pl.ds(row, _R), :], xb, xs).start()

    def wait(slot, ch):
        cb, hb, xb, cs, hs, xs = bufs[slot]
        row = base + ch * _R
        pltpu.make_async_copy(
            c_hbm.at[pl.ds(row, _R), :], cb, cs).wait()
        pltpu.make_async_copy(
            h_hbm.at[pl.ds(row, _R), :], hb, hs).wait()
        pltpu.make_async_copy(
            x_hbm.at[pl.ds(row, _R), :], xb, xs).wait()

    zero = jnp.zeros((16,), jnp.float32)
    lane = lax.iota(jnp.int32, 16)

    dnums = lax.GatherDimensionNumbers(
        offset_dims=(), collapsed_slice_dims=(0,), start_index_map=(0,))

    def allsum(v):
        for k in (1, 2, 4, 8):
            idx = jnp.bitwise_xor(lane, k)
            v = v + lax.gather(v, idx[:, None], dnums, slice_sizes=(1,),
                               mode=lax.GatherScatterMode.PROMISE_IN_BOUNDS)
        return v

    def compute(slot, ch):
        cb, hb, xb = bufs[slot][0], bufs[slot][1], bufs[slot][2]
        accs = (zero,) * (3 * _R)

        def part(dbuf, rowstride, woff, niter, accs):
            def jbody(j, a):
                o = j * 16
                wv0 = w0[pl.ds(woff + o, 16)]
                wv1 = w1[pl.ds(woff + o, 16)]
                wv2 = w2[pl.ds(woff + o, 16)]
                out = []
                for r in range(_R):
                    d = dbuf[r, pl.ds(o, 16)]
                    out.append(a[3 * r] + d * wv0)
                    out.append(a[3 * r + 1] + d * wv1)
                    out.append(a[3 * r + 2] + d * wv2)
                return tuple(out)
            return lax.fori_loop(0, niter, jbody, accs)

        accs = part(cb, _H, 0, _H // 16, accs)
        accs = part(hb, _H, _H, _H // 16, accs)
        accs = part(xb, _X, 2 * _H, _X // 16, accs)

        l0 = zero
        l1 = zero
        l2 = zero
        for r in range(_R):
            s0 = allsum(accs[3 * r])
            s1 = allsum(accs[3 * r + 1])
            s2 = allsum(accs[3 * r + 2])
            sel = lane == r
            l0 = jnp.where(sel, s0, l0)
            l1 = jnp.where(sel, s1, l1)
            l2 = jnp.where(sel, s2, l2)
        l0 = l0 + bv0[...]
        l1 = l1 + bv1[...]
        l2 = l2 + bv2[...]
        m = jnp.maximum(l0, jnp.maximum(l1, l2))
        e0 = jnp.exp(l0 - m)
        e1 = jnp.exp(l1 - m)
        e2 = jnp.exp(l2 - m)
        rec = 1.0 / (e0 + e1 + e2)
        off = ch * _R
        ob0[pl.ds(off, 16)] = e0 * rec
        ob1[pl.ds(off, 16)] = e1 * rec
        ob2[pl.ds(off, 16)] = e2 * rec

    start(0, 0)

    def outer(k, _):
        ch0 = k * 2
        wait(0, ch0)
        start(1, ch0 + 1)
        compute(0, ch0)
        wait(1, ch0 + 1)

        @pl.when(ch0 + 2 < _NCH)
        def _():
            start(0, ch0 + 2)

        compute(1, ch0 + 1)
        return ()

    lax.fori_loop(0, _NCH // 2, outer, ())

    pltpu.make_async_copy(
        ob0.at[pl.ds(0, _RPW)], o0_hbm.at[pl.ds(base, _RPW)], cs0).start()
    pltpu.make_async_copy(
        ob1.at[pl.ds(0, _RPW)], o1_hbm.at[pl.ds(base, _RPW)], hs0).start()
    pltpu.make_async_copy(
        ob2.at[pl.ds(0, _RPW)], o2_hbm.at[pl.ds(base, _RPW)], xs0).start()
    pltpu.make_async_copy(
        ob0.at[pl.ds(0, _RPW)], o0_hbm.at[pl.ds(base, _RPW)], cs0).wait()
    pltpu.make_async_copy(
        ob1.at[pl.ds(0, _RPW)], o1_hbm.at[pl.ds(base, _RPW)], hs0).wait()
    pltpu.make_async_copy(
        ob2.at[pl.ds(0, _RPW)], o2_hbm.at[pl.ds(base, _RPW)], xs0).wait()


def kernel(c_img, h_t, x_t, W, b):
    mesh = plsc.VectorSubcoreMesh(core_axis_name="c", subcore_axis_name="s")
    b0 = jnp.full((16,), b[0], jnp.float32)
    b1 = jnp.full((16,), b[1], jnp.float32)
    b2 = jnp.full((16,), b[2], jnp.float32)
    run = functools.partial(
        pl.kernel, _sc_body, mesh=mesh,
        out_type=[jax.ShapeDtypeStruct((_B,), jnp.float32)] * 3,
        scratch_types=[
            pltpu.VMEM((_R, _H), jnp.float32),
            pltpu.VMEM((_R, _H), jnp.float32),
            pltpu.VMEM((_R, _X), jnp.float32),
            pltpu.VMEM((_R, _H), jnp.float32),
            pltpu.VMEM((_R, _H), jnp.float32),
            pltpu.VMEM((_R, _X), jnp.float32),
            pltpu.VMEM((_D,), jnp.float32),
            pltpu.VMEM((_D,), jnp.float32),
            pltpu.VMEM((_D,), jnp.float32),
            pltpu.VMEM((16,), jnp.float32),
            pltpu.VMEM((16,), jnp.float32),
            pltpu.VMEM((16,), jnp.float32),
            pltpu.VMEM((_RPW + 8,), jnp.float32),
            pltpu.VMEM((_RPW + 8,), jnp.float32),
            pltpu.VMEM((_RPW + 8,), jnp.float32),
            pltpu.SemaphoreType.DMA,
            pltpu.SemaphoreType.DMA,
            pltpu.SemaphoreType.DMA,
            pltpu.SemaphoreType.DMA,
            pltpu.SemaphoreType.DMA,
            pltpu.SemaphoreType.DMA,
            pltpu.SemaphoreType.DMA,
        ],
    )()
    return tuple(run(c_img, h_t, x_t, W[0], W[1], W[2], b0, b1, b2))


# speedup vs baseline: 2.9418x; 2.9418x over previous
import functools

import jax
import jax.numpy as jnp
from jax import lax
from jax.experimental import pallas as pl
from jax.experimental.pallas import tpu as pltpu
from jax.experimental.pallas import tpu_sc as plsc

_B = 16384
_H = 1024
_X = 2624
_NW = 32
_RPW = _B // _NW
_R = 8
_NCH = _RPW // _R
_BLOCK = 1024


def _sc_body(c_hbm, w0_hbm, w1_hbm, w2_hbm,
             p0_hbm, p1_hbm, p2_hbm,
             cb0, cb1, w0, w1, w2, ob0, ob1, ob2,
             cs0, cs1, wsem):
    wid = lax.axis_index("s") * 2 + lax.axis_index("c")
    base = wid * _RPW

    pltpu.async_copy(w0_hbm, w0, wsem).wait()
    pltpu.async_copy(w1_hbm, w1, wsem).wait()
    pltpu.async_copy(w2_hbm, w2, wsem).wait()

    bufs = ((cb0, cs0), (cb1, cs1))

    def start(slot, ch):
        cb, cs = bufs[slot]
        row = base + ch * _R
        pltpu.make_async_copy(c_hbm.at[pl.ds(row, _R), :], cb, cs).start()

    def wait(slot, ch):
        cb, cs = bufs[slot]
        row = base + ch * _R
        pltpu.make_async_copy(c_hbm.at[pl.ds(row, _R), :], cb, cs).wait()

    zero = jnp.zeros((16,), jnp.float32)
    lane = lax.iota(jnp.int32, 16)
    dnums = lax.GatherDimensionNumbers(
        offset_dims=(), collapsed_slice_dims=(0,), start_index_map=(0,))

    def allsum(v):
        for k in (1, 2, 4, 8):
            idx = jnp.bitwise_xor(lane, k)
            v = v + lax.gather(v, idx[:, None], dnums, slice_sizes=(1,),
                               mode=lax.GatherScatterMode.PROMISE_IN_BOUNDS)
        return v

    def compute(slot, ch):
        cb = bufs[slot][0]
        accs = (zero,) * (3 * _R)

        def jbody(j, a):
            o = j * 16
            wv0 = w0[pl.ds(o, 16)]
            wv1 = w1[pl.ds(o, 16)]
            wv2 = w2[pl.ds(o, 16)]
            out = []
            for r in range(_R):
                d = cb[r, pl.ds(o, 16)]
                out.append(a[3 * r] + d * wv0)
                out.append(a[3 * r + 1] + d * wv1)
                out.append(a[3 * r + 2] + d * wv2)
            return tuple(out)

        accs = lax.fori_loop(0, _H // 16, jbody, accs)

        l0 = zero
        l1 = zero
        l2 = zero
        for r in range(_R):
            sel = lane == r
            l0 = jnp.where(sel, allsum(accs[3 * r]), l0)
            l1 = jnp.where(sel, allsum(accs[3 * r + 1]), l1)
            l2 = jnp.where(sel, allsum(accs[3 * r + 2]), l2)
        off = ch * _R
        ob0[pl.ds(off, 16)] = l0
        ob1[pl.ds(off, 16)] = l1
        ob2[pl.ds(off, 16)] = l2

    start(0, 0)

    def outer(k, _):
        ch0 = k * 2
        wait(0, ch0)
        start(1, ch0 + 1)
        compute(0, ch0)
        wait(1, ch0 + 1)

        @pl.when(ch0 + 2 < _NCH)
        def _():
            start(0, ch0 + 2)

        compute(1, ch0 + 1)
        return ()

    lax.fori_loop(0, _NCH // 2, outer, ())

    pltpu.make_async_copy(
        ob0.at[pl.ds(0, _RPW)], p0_hbm.at[pl.ds(base, _RPW)], cs0).start()
    pltpu.make_async_copy(
        ob1.at[pl.ds(0, _RPW)], p1_hbm.at[pl.ds(base, _RPW)], cs1).start()
    pltpu.make_async_copy(
        ob2.at[pl.ds(0, _RPW)], p2_hbm.at[pl.ds(base, _RPW)], wsem).start()
    pltpu.make_async_copy(
        ob0.at[pl.ds(0, _RPW)], p0_hbm.at[pl.ds(base, _RPW)], cs0).wait()
    pltpu.make_async_copy(
        ob1.at[pl.ds(0, _RPW)], p1_hbm.at[pl.ds(base, _RPW)], cs1).wait()
    pltpu.make_async_copy(
        ob2.at[pl.ds(0, _RPW)], p2_hbm.at[pl.ds(base, _RPW)], wsem).wait()


def _sc_partial(c_img, w0, w1, w2):
    mesh = plsc.VectorSubcoreMesh(core_axis_name="c", subcore_axis_name="s")
    run = functools.partial(
        pl.kernel, _sc_body, mesh=mesh,
        out_type=[jax.ShapeDtypeStruct((_B,), jnp.float32)] * 3,
        scratch_types=[
            pltpu.VMEM((_R, _H), jnp.float32),
            pltpu.VMEM((_R, _H), jnp.float32),
            pltpu.VMEM((_H,), jnp.float32),
            pltpu.VMEM((_H,), jnp.float32),
            pltpu.VMEM((_H,), jnp.float32),
            pltpu.VMEM((_RPW + 8,), jnp.float32),
            pltpu.VMEM((_RPW + 8,), jnp.float32),
            pltpu.VMEM((_RPW + 8,), jnp.float32),
            pltpu.SemaphoreType.DMA,
            pltpu.SemaphoreType.DMA,
            pltpu.SemaphoreType.DMA,
        ],
    )()
    return run(c_img, w0, w1, w2)


def _tc_body(h_ref, xt_ref, wh_ref, wx_ref, b_ref, t0_ref, t1_ref, t2_ref):
    logits = jnp.dot(h_ref[...], wh_ref[...], preferred_element_type=jnp.float32)
    lx = jnp.dot(wx_ref[...], xt_ref[...], preferred_element_type=jnp.float32)
    logits += lx.T
    logits += b_ref[...]
    t0_ref[...] = logits[:, 0]
    t1_ref[...] = logits[:, 1]
    t2_ref[...] = logits[:, 2]


def _tc_partial(h_t, xt, W, b):
    wh = W[:, _H:2 * _H].T
    wx = W[:, 2 * _H:]
    b2 = b.reshape(1, 3)
    grid = (_B // _BLOCK,)
    return pl.pallas_call(
        _tc_body,
        grid=grid,
        in_specs=[
            pl.BlockSpec((_BLOCK, _H), lambda i: (i, 0)),
            pl.BlockSpec((_X, _BLOCK), lambda i: (0, i)),
            pl.BlockSpec((_H, 3), lambda i: (0, 0)),
            pl.BlockSpec((3, _X), lambda i: (0, 0)),
            pl.BlockSpec((1, 3), lambda i: (0, 0)),
        ],
        out_specs=[pl.BlockSpec((_BLOCK,), lambda i: (i,))] * 3,
        out_shape=[jax.ShapeDtypeStruct((_B,), jnp.float32)] * 3,
    )(h_t, xt, wh, wx, b2)


def _combine_body(p0, p1, p2, t0, t1, t2, o0, o1, o2):
    l0 = p0[...] + t0[...]
    l1 = p1[...] + t1[...]
    l2 = p2[...] + t2[...]
    m = jnp.maximum(l0, jnp.maximum(l1, l2))
    e0 = jnp.exp(l0 - m)
    e1 = jnp.exp(l1 - m)
    e2 = jnp.exp(l2 - m)
    rec = 1.0 / (e0 + e1 + e2)
    o0[...] = e0 * rec
    o1[...] = e1 * rec
    o2[...] = e2 * rec


def _combine(p0, p1, p2, t0, t1, t2):
    return pl.pallas_call(
        _combine_body,
        out_shape=[jax.ShapeDtypeStruct((_B,), jnp.float32)] * 3,
    )(p0, p1, p2, t0, t1, t2)


def kernel(c_img, h_t, x_t, W, b):
    xt = x_t.T
    p0, p1, p2 = _sc_partial(c_img, W[0, :_H], W[1, :_H], W[2, :_H])
    t0, t1, t2 = _tc_partial(h_t, xt, W, b)
    return tuple(_combine(p0, p1, p2, t0, t1, t2))
